# baseline (speedup 1.0000x reference)
import functools

import jax
import jax.numpy as jnp
from jax import lax
from jax.experimental import pallas as pl
from jax.experimental.pallas import tpu as pltpu
from jax.experimental.pallas import tpu_sc as plsc

BATCH = 4096
SEQ_LEN = 200
EMBED_DIM = 64
TOTAL = BATCH * SEQ_LEN

NUM_CORES = 2
NUM_SUBCORES = 16
NUM_WORKERS = NUM_CORES * NUM_SUBCORES
PER_WORKER = TOTAL // NUM_WORKERS

CHUNK = SEQ_LEN
NUM_CHUNKS = PER_WORKER // CHUNK
IDX_W = 100
IDX_ROWS = CHUNK // IDX_W
IDX_ALL = PER_WORKER // IDX_W
NBUF = 4


def kernel(inputs, pos_table, tok_table):
    idx2d = inputs.reshape(TOTAL // IDX_W, IDX_W).astype(jnp.int32)
    posidx = jnp.arange(CHUNK, dtype=jnp.int32).reshape(IDX_ROWS, IDX_W)

    mesh = plsc.VectorSubcoreMesh(core_axis_name="c", subcore_axis_name="s")

    @functools.partial(
        pl.kernel,
        out_type=jax.ShapeDtypeStruct((BATCH, SEQ_LEN, EMBED_DIM), jnp.float32),
        mesh=mesh,
        scratch_types=[
            pltpu.VMEM((IDX_ALL, IDX_W), jnp.int32),
            pltpu.VMEM((IDX_ROWS, IDX_W), jnp.int32),
            [pltpu.VMEM((CHUNK, EMBED_DIM), jnp.float32)
             for _ in range(NBUF)],
            pltpu.VMEM_SHARED((SEQ_LEN, EMBED_DIM), jnp.float32),
            [pltpu.SemaphoreType.DMA for _ in range(NBUF)],
            [pltpu.SemaphoreType.DMA for _ in range(NBUF)],
            [pltpu.SemaphoreType.DMA for _ in range(NBUF)],
        ],
        compiler_params=pltpu.CompilerParams(use_tc_tiling_on_sc=False),
    )
    def embed(idx_hbm, posidx_hbm, pos_hbm, tok_hbm, out_hbm,
              idx_v, posidx_v, rows, pos_sh, gsem, psem, ssem):
        wid = lax.axis_index("s") * NUM_CORES + lax.axis_index("c")
        seq_base = wid * NUM_CHUNKS
        idx_base = wid * IDX_ALL

        @pl.when(lax.axis_index("s") == 0)
        def _():
            pltpu.sync_copy(pos_hbm, rows[0])
            pltpu.sync_copy(rows[0], pos_sh)

        pltpu.sync_copy(posidx_hbm, posidx_v)
        pltpu.sync_copy(idx_hbm.at[pl.ds(idx_base, IDX_ALL)], idx_v)
        plsc.subcore_barrier()

        def start_g(c, b):
            for j in range(IDX_ROWS):
                pltpu.async_copy(tok_hbm.at[idx_v.at[c * IDX_ROWS + j]],
                                 rows[b].at[pl.ds(j * IDX_W, IDX_W)], gsem[b])

        def wait_g(b):
            for j in range(IDX_ROWS):
                pltpu.make_async_copy(tok_hbm.at[idx_v.at[j]],
                                      rows[b].at[pl.ds(j * IDX_W, IDX_W)],
                                      gsem[b]).wait()

        def start_p(b):
            for j in range(IDX_ROWS):
                pltpu.async_copy(pos_sh.at[posidx_v.at[j]],
                                 rows[b].at[pl.ds(j * IDX_W, IDX_W)], psem[b],
                                 add=True)

        def wait_p(b):
            for j in range(IDX_ROWS):
                pltpu.make_async_copy(pos_sh.at[posidx_v.at[j]],
                                      rows[b].at[pl.ds(j * IDX_W, IDX_W)],
                                      psem[b]).wait()

        def start_s(c, b):
            pltpu.async_copy(rows[b], out_hbm.at[seq_base + c], ssem[b])

        def wait_s(b):
            pltpu.make_async_copy(rows[b], out_hbm.at[seq_base],
                                  ssem[b]).wait()

        for c in range(3):
            start_g(c, c)
        wait_g(0)
        start_p(0)

        @pl.loop(0, NUM_CHUNKS, step=NBUF)
        def _(cc):
            for b in range(NBUF):
                c = cc + b
                b1 = (b + 1) % NBUF

                @pl.when(c + 1 < NUM_CHUNKS)
                def _():
                    wait_g(b1)
                    start_p(b1)

                wait_p(b)
                start_s(c, b)

                b3 = (b + 3) % NBUF

                @pl.when(c + 3 < NUM_CHUNKS)
                def _():
                    @pl.when(c >= 1)
                    def _():
                        wait_s(b3)

                    start_g(c + 3, b3)

        for b in range(NBUF):
            wait_s(b)

    return embed(idx2d, posidx, pos_table, tok_table)
